# baseline (speedup 1.0000x reference)
import functools

import jax
import jax.numpy as jnp
from jax import lax
from jax.experimental import pallas as pl
from jax.experimental.pallas import tpu as pltpu
from jax.experimental.pallas import tpu_sc as plsc

HID = 768
NCHUNK = HID // 16
ANS_NUM = 5000
OCR_NUM = 50
BATCH = 32
DEC_LEN = 100
RPC = 16
NCH = 7


def _rsqrt(x):
    xi = lax.bitcast_convert_type(x, jnp.int32)
    yi = jnp.int32(0x5F3759DF) - lax.shift_right_arithmetic(xi, 1)
    y = lax.bitcast_convert_type(yi, jnp.float32)
    for _ in range(3):
        y = y * (1.5 - 0.5 * x * y * y)
    return y


_GATHER_DNUMS = lax.GatherDimensionNumbers(
    offset_dims=(), collapsed_slice_dims=(0,), start_index_map=(0,))


def _permute(v, idx):
    return lax.gather(v, idx[:, None], _GATHER_DNUMS, slice_sizes=(1,),
                      mode=lax.GatherScatterMode.PROMISE_IN_BOUNDS)


def _lane_total(v):
    i = lax.iota(jnp.int32, 16)
    for st in (1, 2, 4, 8):
        v = v + _permute(v, i ^ st)
    return v


def _row_stats(read):
    zero = jnp.zeros((16,), jnp.float32)
    s = [zero] * 4
    s2 = [zero] * 4
    for j in range(NCHUNK):
        x = read(j)
        k = j % 4
        s[k] = s[k] + x
        s2[k] = s2[k] + x * x
    mu = _lane_total((s[0] + s[1]) + (s[2] + s[3])) * (1.0 / HID)
    var = _lane_total((s2[0] + s2[1]) + (s2[2] + s2[3])) * (1.0 / HID) - mu * mu
    return mu, _rsqrt(var + 1e-12)


def _sc_body(ans_hbm, ocr_hbm, prev_hbm, tt_hbm,
             ans_w_hbm, ans_b_hbm, ocr_w_hbm, ocr_b_hbm,
             emb_w_hbm, emb_b_hbm, out_hbm,
             idx_v, aidx_v, oidx_v, didx_v, rows_v, stage, tt_v, pwt, pbt,
             ew, eb, sem0, sem1):
    nc = 2
    wid = lax.axis_index("s") * nc + lax.axis_index("c")
    iota = lax.iota(jnp.int32, 16)
    zeros_i = jnp.zeros((16,), jnp.int32)

    tok0 = wid * DEC_LEN
    r8 = lax.bitwise_and(tok0, 7)
    idx_v[pl.ds(96, 16)] = zeros_i
    idx_v[pl.ds(112, 16)] = zeros_i
    abase = pl.multiple_of(tok0 - r8, 8)
    pltpu.sync_copy(prev_hbm.at[pl.ds(abase, 104)], idx_v.at[pl.ds(0, 104)])

    for k in range(NCH):
        v = plsc.load_gather(idx_v, [jnp.broadcast_to(r8 + k * 16, (16,)) + iota])
        t = v >= ANS_NUM
        aidx_v[pl.ds(k * 16, 16)] = jnp.where(t, 0, v)
        oidx_v[pl.ds(k * 16, 16)] = jnp.where(t, (v - ANS_NUM) * BATCH + wid, wid)
        tokv = iota + (k * 16)
        if k == NCH - 1:
            tokv = jnp.where(iota < 4, tokv, iota - 4)
        didx_v[k, :] = tokv * BATCH + wid

    pltpu.sync_copy(ans_w_hbm, pwt.at[0])
    pltpu.sync_copy(ocr_w_hbm, pwt.at[1])
    pltpu.sync_copy(ans_b_hbm, pbt.at[0])
    pltpu.sync_copy(ocr_b_hbm, pbt.at[1])
    pltpu.sync_copy(emb_w_hbm, ew)
    pltpu.sync_copy(emb_b_hbm, eb)
    pltpu.sync_copy(tt_hbm.at[pl.ds(0, 2 * HID)], tt_v)

    mu0, rs0 = _row_stats(lambda j: tt_v[pl.ds(j * 16, 16)])
    mu1, rs1 = _row_stats(lambda j: tt_v[pl.ds(HID + j * 16, 16)])
    for j in range(NCHUNK):
        sl = pl.ds(j * 16, 16)
        tte0 = (tt_v[sl] - mu0) * rs0 * ew[sl] + eb[sl]
        pbt[0, sl] = pbt[0, sl] + tte0
        tte1 = (tt_v[pl.ds(HID + j * 16, 16)] - mu1) * rs1 * ew[sl] + eb[sl]
        pbt[1, sl] = pbt[1, sl] + tte1

    def issue(ck, slot):
        sem = sem0 if slot == 0 else sem1
        sl = pl.ds(ck * RPC, RPC)
        ca = pltpu.async_copy(ans_hbm.at[aidx_v.at[sl]],
                              rows_v.at[pl.ds(slot * 32, RPC)], sem)
        co = pltpu.async_copy(ocr_hbm.at[oidx_v.at[sl]],
                              rows_v.at[pl.ds(slot * 32 + RPC, RPC)], sem)
        return ca, co

    def compute_chunk(ck, slot):
        vidx = plsc.load_gather(
            idx_v, [jnp.broadcast_to(r8 + ck * RPC, (16,)) + iota])
        t_i32 = jnp.where(vidx >= ANS_NUM, 1, 0)
        rowvec = iota + (slot * 32) + t_i32 * RPC

        zero = jnp.zeros((16,), jnp.float32)

        def stat_body(j, carry):
            s = list(carry[:4])
            s2 = list(carry[4:])
            base = j * 16
            for u in range(16):
                ev = jnp.broadcast_to(base + u, (16,))
                x = plsc.load_gather(rows_v, [rowvec, ev])
                s[u % 4] = s[u % 4] + x
                s2[u % 4] = s2[u % 4] + x * x
            return tuple(s) + tuple(s2)

        acc = lax.fori_loop(0, NCHUNK, stat_body, (zero,) * 8)
        mu = ((acc[0] + acc[1]) + (acc[2] + acc[3])) * (1.0 / HID)
        var = ((acc[4] + acc[5]) + (acc[6] + acc[7])) * (1.0 / HID) - mu * mu
        rs = _rsqrt(var + 1e-12)
        murs = mu * rs

        def norm_body(j, carry):
            base = j * 16
            for u in range(16):
                ev = jnp.broadcast_to(base + u, (16,))
                x = plsc.load_gather(rows_v, [rowvec, ev])
                wv = plsc.load_gather(pwt, [t_i32, ev])
                bv = plsc.load_gather(pbt, [t_i32, ev])
                z = x * rs - murs
                plsc.store_scatter(stage, [iota, ev], z * wv + bv)
            return carry

        lax.fori_loop(0, NCHUNK, norm_body, 0)
        pltpu.sync_copy(stage, out_hbm.at[didx_v.at[ck]])

    order = [NCH - 1] + list(range(NCH - 1))
    pending = {0: issue(order[0], 0), 1: issue(order[1], 1)}
    for i, ck in enumerate(order):
        slot = i % 2
        ca, co = pending.pop(i)
        ca.wait()
        co.wait()
        compute_chunk(ck, slot)
        if i + 2 < NCH:
            pending[i + 2] = issue(order[i + 2], slot)


def kernel(ans_emb, ocr_emb, prev_inds, ans_w, ans_b, ocr_w, ocr_b, emb_w, emb_b, tt_table):
    batch = ocr_emb.shape[0]
    ocr_flat = jnp.transpose(ocr_emb, (1, 0, 2)).reshape(OCR_NUM * batch, HID)
    prev_flat = prev_inds.reshape(-1)
    tt_flat = tt_table.reshape(-1)
    mesh = plsc.VectorSubcoreMesh(core_axis_name="c", subcore_axis_name="s")
    run = functools.partial(
        pl.kernel,
        mesh=mesh,
        compiler_params=pltpu.CompilerParams(needs_layout_passes=False),
        out_type=jax.ShapeDtypeStruct((DEC_LEN * batch, HID), jnp.float32),
        scratch_types=[
            pltpu.VMEM((128,), jnp.int32),
            pltpu.VMEM((NCH * RPC,), jnp.int32),
            pltpu.VMEM((NCH * RPC,), jnp.int32),
            pltpu.VMEM((NCH, RPC), jnp.int32),
            pltpu.VMEM((64, HID), jnp.float32),
            pltpu.VMEM((RPC, HID), jnp.float32),
            pltpu.VMEM((2 * HID,), jnp.float32),
            pltpu.VMEM((2, HID), jnp.float32),
            pltpu.VMEM((2, HID), jnp.float32),
            pltpu.VMEM((HID,), jnp.float32),
            pltpu.VMEM((HID,), jnp.float32),
            pltpu.SemaphoreType.DMA,
            pltpu.SemaphoreType.DMA,
        ],
    )(_sc_body)
    out = run(ans_emb, ocr_flat, prev_flat, tt_flat,
              ans_w, ans_b, ocr_w, ocr_b, emb_w, emb_b)
    return jnp.transpose(out.reshape(DEC_LEN, batch, HID), (1, 0, 2))


# speedup vs baseline: 1.0287x; 1.0287x over previous
import functools

import jax
import jax.numpy as jnp
from jax import lax
from jax.experimental import pallas as pl
from jax.experimental.pallas import tpu as pltpu
from jax.experimental.pallas import tpu_sc as plsc

HID = 768
NCHUNK = HID // 16
ANS_NUM = 5000
OCR_NUM = 50
BATCH = 32
DEC_LEN = 100
RPC = 16
NCH = 7


def _rsqrt(x):
    xi = lax.bitcast_convert_type(x, jnp.int32)
    yi = jnp.int32(0x5F3759DF) - lax.shift_right_arithmetic(xi, 1)
    y = lax.bitcast_convert_type(yi, jnp.float32)
    for _ in range(3):
        y = y * (1.5 - 0.5 * x * y * y)
    return y


_GATHER_DNUMS = lax.GatherDimensionNumbers(
    offset_dims=(), collapsed_slice_dims=(0,), start_index_map=(0,))


def _permute(v, idx):
    return lax.gather(v, idx[:, None], _GATHER_DNUMS, slice_sizes=(1,),
                      mode=lax.GatherScatterMode.PROMISE_IN_BOUNDS)


def _lane_total(v):
    i = lax.iota(jnp.int32, 16)
    for st in (1, 2, 4, 8):
        v = v + _permute(v, i ^ st)
    return v


def _row_stats(read):
    zero = jnp.zeros((16,), jnp.float32)
    s = [zero] * 4
    s2 = [zero] * 4
    for j in range(NCHUNK):
        x = read(j)
        k = j % 4
        s[k] = s[k] + x
        s2[k] = s2[k] + x * x
    mu = _lane_total((s[0] + s[1]) + (s[2] + s[3])) * (1.0 / HID)
    var = _lane_total((s2[0] + s2[1]) + (s2[2] + s2[3])) * (1.0 / HID) - mu * mu
    return mu, _rsqrt(var + 1e-12)


def _sc_body(ans_hbm, ocr_hbm, prev_hbm, tt_hbm,
             ans_w_hbm, ans_b_hbm, ocr_w_hbm, ocr_b_hbm,
             emb_w_hbm, emb_b_hbm, out_hbm,
             idx_v, aidx_v, oidx_v, didx_v, rows_v, stage, tt_v, pwt, pbt,
             ew, eb, sem0, sem1):
    nc = 2
    wid = lax.axis_index("s") * nc + lax.axis_index("c")
    iota = lax.iota(jnp.int32, 16)
    zeros_i = jnp.zeros((16,), jnp.int32)

    tok0 = wid * DEC_LEN
    r8 = lax.bitwise_and(tok0, 7)
    idx_v[pl.ds(96, 16)] = zeros_i
    idx_v[pl.ds(112, 16)] = zeros_i
    abase = pl.multiple_of(tok0 - r8, 8)
    pltpu.sync_copy(prev_hbm.at[pl.ds(abase, 104)], idx_v.at[pl.ds(0, 104)])

    for k in range(NCH):
        v = plsc.load_gather(idx_v, [jnp.broadcast_to(r8 + k * 16, (16,)) + iota])
        t = v >= ANS_NUM
        aidx_v[pl.ds(k * 16, 16)] = jnp.where(t, 0, v)
        oidx_v[pl.ds(k * 16, 16)] = jnp.where(t, (v - ANS_NUM) * BATCH + wid, wid)
        tokv = iota + (k * 16)
        if k == NCH - 1:
            tokv = jnp.where(iota < 4, tokv, iota - 4)
        didx_v[k, :] = tokv * BATCH + wid

    pltpu.sync_copy(ans_w_hbm, pwt.at[pl.ds(0, HID)])
    pltpu.sync_copy(ocr_w_hbm, pwt.at[pl.ds(HID, HID)])
    pltpu.sync_copy(ans_b_hbm, pbt.at[pl.ds(0, HID)])
    pltpu.sync_copy(ocr_b_hbm, pbt.at[pl.ds(HID, HID)])
    pltpu.sync_copy(emb_w_hbm, ew)
    pltpu.sync_copy(emb_b_hbm, eb)
    pltpu.sync_copy(tt_hbm.at[pl.ds(0, 2 * HID)], tt_v)

    mu0, rs0 = _row_stats(lambda j: tt_v[pl.ds(j * 16, 16)])
    mu1, rs1 = _row_stats(lambda j: tt_v[pl.ds(HID + j * 16, 16)])
    for j in range(NCHUNK):
        sl = pl.ds(j * 16, 16)
        sl2 = pl.ds(HID + j * 16, 16)
        tte0 = (tt_v[sl] - mu0) * rs0 * ew[sl] + eb[sl]
        pbt[sl] = pbt[sl] + tte0
        tte1 = (tt_v[sl2] - mu1) * rs1 * ew[sl] + eb[sl]
        pbt[sl2] = pbt[sl2] + tte1

    def issue(ck, slot):
        sem = sem0 if slot == 0 else sem1
        sl = pl.ds(ck * RPC, RPC)
        ca = pltpu.async_copy(ans_hbm.at[aidx_v.at[sl]],
                              rows_v.at[pl.ds(slot * 32, RPC), pl.ds(0, HID)],
                              sem)
        co = pltpu.async_copy(ocr_hbm.at[oidx_v.at[sl]],
                              rows_v.at[pl.ds(slot * 32 + RPC, RPC), pl.ds(0, HID)],
                              sem)
        return ca, co

    def compute_chunk(ck, slot):
        vidx = plsc.load_gather(
            idx_v, [jnp.broadcast_to(r8 + ck * RPC, (16,)) + iota])
        t_i32 = jnp.where(vidx >= ANS_NUM, 1, 0)
        rowvec = iota + (slot * 32) + t_i32 * RPC

        zero = jnp.zeros((16,), jnp.float32)
        one = jnp.full((16,), 1, jnp.int32)

        def stat_body(j, carry):
            s = list(carry[:4])
            s2 = list(carry[4:8])
            cs = carry[8]
            for u in range(16):
                x = plsc.load_gather(rows_v, [rowvec, cs])
                cs = cs + one
                s[u % 4] = s[u % 4] + x
                s2[u % 4] = s2[u % 4] + x * x
            return tuple(s) + tuple(s2) + (cs,)

        acc = lax.fori_loop(0, NCHUNK, stat_body, (zero,) * 8 + (zeros_i,))
        mu = ((acc[0] + acc[1]) + (acc[2] + acc[3])) * (1.0 / HID)
        var = ((acc[4] + acc[5]) + (acc[6] + acc[7])) * (1.0 / HID) - mu * mu
        rs = _rsqrt(var + 1e-12)
        murs = mu * rs

        pbase = t_i32 * HID

        def norm_body(j, carry):
            cs, pidx = carry
            for u in range(16):
                x = plsc.load_gather(rows_v, [rowvec, cs])
                wv = plsc.load_gather(pwt, [pidx])
                bv = plsc.load_gather(pbt, [pidx])
                z = x * rs - murs
                plsc.store_scatter(stage, [iota, cs], z * wv + bv)
                cs = cs + one
                pidx = pidx + one
            return cs, pidx

        lax.fori_loop(0, NCHUNK, norm_body, (zeros_i, pbase))
        pltpu.sync_copy(stage.at[pl.ds(0, RPC), pl.ds(0, HID)],
                        out_hbm.at[didx_v.at[ck]])

    order = [NCH - 1] + list(range(NCH - 1))
    pending = {0: issue(order[0], 0), 1: issue(order[1], 1)}
    for i, ck in enumerate(order):
        slot = i % 2
        ca, co = pending.pop(i)
        ca.wait()
        co.wait()
        compute_chunk(ck, slot)
        if i + 2 < NCH:
            pending[i + 2] = issue(order[i + 2], slot)


def kernel(ans_emb, ocr_emb, prev_inds, ans_w, ans_b, ocr_w, ocr_b, emb_w, emb_b, tt_table):
    batch = ocr_emb.shape[0]
    ocr_flat = jnp.transpose(ocr_emb, (1, 0, 2)).reshape(OCR_NUM * batch, HID)
    prev_flat = prev_inds.reshape(-1)
    tt_flat = tt_table.reshape(-1)
    mesh = plsc.VectorSubcoreMesh(core_axis_name="c", subcore_axis_name="s")
    run = functools.partial(
        pl.kernel,
        mesh=mesh,
        compiler_params=pltpu.CompilerParams(needs_layout_passes=False),
        out_type=jax.ShapeDtypeStruct((DEC_LEN * batch, HID), jnp.float32),
        scratch_types=[
            pltpu.VMEM((128,), jnp.int32),
            pltpu.VMEM((NCH * RPC,), jnp.int32),
            pltpu.VMEM((NCH * RPC,), jnp.int32),
            pltpu.VMEM((NCH, RPC), jnp.int32),
            pltpu.VMEM((64, 1024), jnp.float32),
            pltpu.VMEM((RPC, 1024), jnp.float32),
            pltpu.VMEM((2 * HID,), jnp.float32),
            pltpu.VMEM((2 * HID,), jnp.float32),
            pltpu.VMEM((2 * HID,), jnp.float32),
            pltpu.VMEM((HID,), jnp.float32),
            pltpu.VMEM((HID,), jnp.float32),
            pltpu.SemaphoreType.DMA,
            pltpu.SemaphoreType.DMA,
        ],
    )(_sc_body)
    out = run(ans_emb, ocr_flat, prev_flat, tt_flat,
              ans_w, ans_b, ocr_w, ocr_b, emb_w, emb_b)
    return jnp.transpose(out.reshape(DEC_LEN, batch, HID), (1, 0, 2))


# speedup vs baseline: 1.4943x; 1.4527x over previous
import functools

import jax
import jax.numpy as jnp
from jax import lax
from jax.experimental import pallas as pl
from jax.experimental.pallas import tpu as pltpu
from jax.experimental.pallas import tpu_sc as plsc

HID = 768
NCHUNK = HID // 16
ANS_NUM = 5000
OCR_NUM = 50
BATCH = 32
DEC_LEN = 100
RPC = 16
NCH = 7


def _rsqrt(x):
    xi = lax.bitcast_convert_type(x, jnp.int32)
    yi = jnp.int32(0x5F3759DF) - lax.shift_right_arithmetic(xi, 1)
    y = lax.bitcast_convert_type(yi, jnp.float32)
    for _ in range(3):
        y = y * (1.5 - 0.5 * x * y * y)
    return y


_GATHER_DNUMS = lax.GatherDimensionNumbers(
    offset_dims=(), collapsed_slice_dims=(0,), start_index_map=(0,))


def _permute(v, idx):
    return lax.gather(v, idx[:, None], _GATHER_DNUMS, slice_sizes=(1,),
                      mode=lax.GatherScatterMode.PROMISE_IN_BOUNDS)


def _lane_total(v):
    i = lax.iota(jnp.int32, 16)
    for st in (1, 2, 4, 8):
        v = v + _permute(v, i ^ st)
    return v


def _row_stats(read):
    zero = jnp.zeros((16,), jnp.float32)
    s = [zero] * 4
    s2 = [zero] * 4
    for j in range(NCHUNK):
        x = read(j)
        k = j % 4
        s[k] = s[k] + x
        s2[k] = s2[k] + x * x
    mu = _lane_total((s[0] + s[1]) + (s[2] + s[3])) * (1.0 / HID)
    var = _lane_total((s2[0] + s2[1]) + (s2[2] + s2[3])) * (1.0 / HID) - mu * mu
    return mu, _rsqrt(var + 1e-12)


def _sc_body(ans_hbm, ocr_hbm, prev_hbm, tt_hbm,
             ans_w_hbm, ans_b_hbm, ocr_w_hbm, ocr_b_hbm,
             emb_w_hbm, emb_b_hbm, out_hbm,
             idx_v, aidx_v, oidx_v, didx_v, rows_v, stage, tt_v, pwt, pbt,
             ew, eb, sem0, sem1):
    nc = 2
    wid = lax.axis_index("s") * nc + lax.axis_index("c")
    iota = lax.iota(jnp.int32, 16)
    zeros_i = jnp.zeros((16,), jnp.int32)

    tok0 = wid * DEC_LEN
    r8 = lax.bitwise_and(tok0, 7)
    idx_v[pl.ds(96, 16)] = zeros_i
    idx_v[pl.ds(112, 16)] = zeros_i
    abase = pl.multiple_of(tok0 - r8, 8)
    pltpu.sync_copy(prev_hbm.at[pl.ds(abase, 104)], idx_v.at[pl.ds(0, 104)])

    for k in range(NCH):
        v = plsc.load_gather(idx_v, [jnp.broadcast_to(r8 + k * 16, (16,)) + iota])
        t = v >= ANS_NUM
        aidx_v[pl.ds(k * 16, 16)] = jnp.where(t, 0, v)
        oidx_v[pl.ds(k * 16, 16)] = jnp.where(t, (v - ANS_NUM) * BATCH + wid, wid)
        tokv = iota + (k * 16)
        if k == NCH - 1:
            tokv = jnp.where(iota < 4, tokv, iota - 4)
        didx_v[k, :] = tokv * BATCH + wid

    pltpu.sync_copy(ans_w_hbm, pwt.at[pl.ds(0, HID)])
    pltpu.sync_copy(ocr_w_hbm, pwt.at[pl.ds(HID, HID)])
    pltpu.sync_copy(ans_b_hbm, pbt.at[pl.ds(0, HID)])
    pltpu.sync_copy(ocr_b_hbm, pbt.at[pl.ds(HID, HID)])
    pltpu.sync_copy(emb_w_hbm, ew)
    pltpu.sync_copy(emb_b_hbm, eb)
    pltpu.sync_copy(tt_hbm.at[pl.ds(0, 2 * HID)], tt_v)

    mu0, rs0 = _row_stats(lambda j: tt_v[pl.ds(j * 16, 16)])
    mu1, rs1 = _row_stats(lambda j: tt_v[pl.ds(HID + j * 16, 16)])
    for j in range(NCHUNK):
        sl = pl.ds(j * 16, 16)
        sl2 = pl.ds(HID + j * 16, 16)
        tte0 = (tt_v[sl] - mu0) * rs0 * ew[sl] + eb[sl]
        pbt[sl] = pbt[sl] + tte0
        tte1 = (tt_v[sl2] - mu1) * rs1 * ew[sl] + eb[sl]
        pbt[sl2] = pbt[sl2] + tte1

    def issue(ck, slot):
        sem = sem0 if slot == 0 else sem1
        sl = pl.ds(ck * RPC, RPC)
        ca = pltpu.async_copy(ans_hbm.at[aidx_v.at[sl]],
                              rows_v.at[pl.ds(slot * 32, RPC), pl.ds(0, HID)],
                              sem)
        co = pltpu.async_copy(ocr_hbm.at[oidx_v.at[sl]],
                              rows_v.at[pl.ds(slot * 32 + RPC, RPC), pl.ds(0, HID)],
                              sem)
        return ca, co

    def compute_chunk(ck, slot):
        vidx = plsc.load_gather(
            idx_v, [jnp.broadcast_to(r8 + ck * RPC, (16,)) + iota])
        t_i32 = jnp.where(vidx >= ANS_NUM, 1, 0)
        rowvec = iota + (slot * 32) + t_i32 * RPC

        zero = jnp.zeros((16,), jnp.float32)
        one = jnp.full((16,), 1, jnp.int32)

        lane16 = iota * 16

        def wrap_inc(cs):
            cs1 = cs + one
            return jnp.where(cs1 >= HID, cs1 - HID, cs1)

        def stat_body(j, carry):
            s = list(carry[:4])
            s2 = list(carry[4:8])
            cs = carry[8]
            for u in range(16):
                x = plsc.load_gather(rows_v, [rowvec, cs])
                cs = wrap_inc(cs)
                s[u % 4] = s[u % 4] + x
                s2[u % 4] = s2[u % 4] + x * x
            return tuple(s) + tuple(s2) + (cs,)

        acc = lax.fori_loop(0, NCHUNK, stat_body, (zero,) * 8 + (lane16,))
        mu = ((acc[0] + acc[1]) + (acc[2] + acc[3])) * (1.0 / HID)
        var = ((acc[4] + acc[5]) + (acc[6] + acc[7])) * (1.0 / HID) - mu * mu
        rs = _rsqrt(var + 1e-12)
        murs = mu * rs

        pbase = t_i32 * HID

        def norm_body(j, cs):
            for u in range(16):
                x = plsc.load_gather(rows_v, [rowvec, cs])
                pidx = pbase + cs
                wv = plsc.load_gather(pwt, [pidx])
                bv = plsc.load_gather(pbt, [pidx])
                z = x * rs - murs
                plsc.store_scatter(stage, [iota, cs], z * wv + bv)
                cs = wrap_inc(cs)
            return cs

        lax.fori_loop(0, NCHUNK, norm_body, lane16)
        pltpu.sync_copy(stage.at[pl.ds(0, RPC), pl.ds(0, HID)],
                        out_hbm.at[didx_v.at[ck]])

    order = [NCH - 1] + list(range(NCH - 1))
    pending = {0: issue(order[0], 0), 1: issue(order[1], 1)}
    for i, ck in enumerate(order):
        slot = i % 2
        ca, co = pending.pop(i)
        ca.wait()
        co.wait()
        compute_chunk(ck, slot)
        if i + 2 < NCH:
            pending[i + 2] = issue(order[i + 2], slot)


def kernel(ans_emb, ocr_emb, prev_inds, ans_w, ans_b, ocr_w, ocr_b, emb_w, emb_b, tt_table):
    batch = ocr_emb.shape[0]
    ocr_flat = jnp.transpose(ocr_emb, (1, 0, 2)).reshape(OCR_NUM * batch, HID)
    prev_flat = prev_inds.reshape(-1)
    tt_flat = tt_table.reshape(-1)
    mesh = plsc.VectorSubcoreMesh(core_axis_name="c", subcore_axis_name="s")
    run = functools.partial(
        pl.kernel,
        mesh=mesh,
        compiler_params=pltpu.CompilerParams(needs_layout_passes=False),
        out_type=jax.ShapeDtypeStruct((DEC_LEN * batch, HID), jnp.float32),
        scratch_types=[
            pltpu.VMEM((128,), jnp.int32),
            pltpu.VMEM((NCH * RPC,), jnp.int32),
            pltpu.VMEM((NCH * RPC,), jnp.int32),
            pltpu.VMEM((NCH, RPC), jnp.int32),
            pltpu.VMEM((64, 1024), jnp.float32),
            pltpu.VMEM((RPC, 1024), jnp.float32),
            pltpu.VMEM((2 * HID,), jnp.float32),
            pltpu.VMEM((2 * HID,), jnp.float32),
            pltpu.VMEM((2 * HID,), jnp.float32),
            pltpu.VMEM((HID,), jnp.float32),
            pltpu.VMEM((HID,), jnp.float32),
            pltpu.SemaphoreType.DMA,
            pltpu.SemaphoreType.DMA,
        ],
    )(_sc_body)
    out = run(ans_emb, ocr_flat, prev_flat, tt_flat,
              ans_w, ans_b, ocr_w, ocr_b, emb_w, emb_b)
    return jnp.transpose(out.reshape(DEC_LEN, batch, HID), (1, 0, 2))


# speedup vs baseline: 1.5536x; 1.0397x over previous
import functools

import jax
import jax.numpy as jnp
from jax import lax
from jax.experimental import pallas as pl
from jax.experimental.pallas import tpu as pltpu
from jax.experimental.pallas import tpu_sc as plsc

HID = 768
NCHUNK = HID // 16
ANS_NUM = 5000
OCR_NUM = 50
BATCH = 32
DEC_LEN = 100
RPC = 16
NCH = 7


def _rsqrt(x):
    xi = lax.bitcast_convert_type(x, jnp.int32)
    yi = jnp.int32(0x5F3759DF) - lax.shift_right_arithmetic(xi, 1)
    y = lax.bitcast_convert_type(yi, jnp.float32)
    for _ in range(3):
        y = y * (1.5 - 0.5 * x * y * y)
    return y


_GATHER_DNUMS = lax.GatherDimensionNumbers(
    offset_dims=(), collapsed_slice_dims=(0,), start_index_map=(0,))


def _permute(v, idx):
    return lax.gather(v, idx[:, None], _GATHER_DNUMS, slice_sizes=(1,),
                      mode=lax.GatherScatterMode.PROMISE_IN_BOUNDS)


def _lane_total(v):
    i = lax.iota(jnp.int32, 16)
    for st in (1, 2, 4, 8):
        v = v + _permute(v, i ^ st)
    return v


def _row_stats(read):
    zero = jnp.zeros((16,), jnp.float32)
    s = [zero] * 4
    s2 = [zero] * 4
    for j in range(NCHUNK):
        x = read(j)
        k = j % 4
        s[k] = s[k] + x
        s2[k] = s2[k] + x * x
    mu = _lane_total((s[0] + s[1]) + (s[2] + s[3])) * (1.0 / HID)
    var = _lane_total((s2[0] + s2[1]) + (s2[2] + s2[3])) * (1.0 / HID) - mu * mu
    return mu, _rsqrt(var + 1e-12)


def _sc_body(ans_hbm, ocr_hbm, prev_hbm, tt_hbm,
             ans_w_hbm, ans_b_hbm, ocr_w_hbm, ocr_b_hbm,
             emb_w_hbm, emb_b_hbm, out_hbm,
             idx_v, aidx_v, oidx_v, didx_v, rows_v, stage, tt_v, pwt, pbt,
             ew, eb, sem0, sem1):
    nc = 2
    wid = lax.axis_index("s") * nc + lax.axis_index("c")
    iota = lax.iota(jnp.int32, 16)
    zeros_i = jnp.zeros((16,), jnp.int32)

    tok0 = wid * DEC_LEN
    r8 = lax.bitwise_and(tok0, 7)
    idx_v[pl.ds(96, 16)] = zeros_i
    idx_v[pl.ds(112, 16)] = zeros_i
    abase = pl.multiple_of(tok0 - r8, 8)
    pltpu.sync_copy(prev_hbm.at[pl.ds(abase, 104)], idx_v.at[pl.ds(0, 104)])

    for k in range(NCH):
        v = plsc.load_gather(idx_v, [jnp.broadcast_to(r8 + k * 16, (16,)) + iota])
        t = v >= ANS_NUM
        aidx_v[pl.ds(k * 16, 16)] = jnp.where(t, 0, v)
        oidx_v[pl.ds(k * 16, 16)] = jnp.where(t, (v - ANS_NUM) * BATCH + wid, wid)
        tokv = iota + (k * 16)
        if k == NCH - 1:
            tokv = jnp.where(iota < 4, tokv, iota - 4)
        didx_v[k, :] = tokv * BATCH + wid

    pltpu.sync_copy(ans_w_hbm, pwt.at[pl.ds(0, HID)])
    pltpu.sync_copy(ocr_w_hbm, pwt.at[pl.ds(HID, HID)])
    pltpu.sync_copy(ans_b_hbm, pbt.at[pl.ds(0, HID)])
    pltpu.sync_copy(ocr_b_hbm, pbt.at[pl.ds(HID, HID)])
    pltpu.sync_copy(emb_w_hbm, ew)
    pltpu.sync_copy(emb_b_hbm, eb)
    pltpu.sync_copy(tt_hbm.at[pl.ds(0, 2 * HID)], tt_v)

    mu0, rs0 = _row_stats(lambda j: tt_v[pl.ds(j * 16, 16)])
    mu1, rs1 = _row_stats(lambda j: tt_v[pl.ds(HID + j * 16, 16)])
    for j in range(NCHUNK):
        sl = pl.ds(j * 16, 16)
        sl2 = pl.ds(HID + j * 16, 16)
        tte0 = (tt_v[sl] - mu0) * rs0 * ew[sl] + eb[sl]
        pbt[sl] = pbt[sl] + tte0
        tte1 = (tt_v[sl2] - mu1) * rs1 * ew[sl] + eb[sl]
        pbt[sl2] = pbt[sl2] + tte1

    def issue(ck, sbase, sem):
        sl = pl.ds(ck * RPC, RPC)
        pltpu.async_copy(ans_hbm.at[aidx_v.at[sl]],
                         rows_v.at[pl.ds(sbase, RPC), pl.ds(0, HID)], sem)
        pltpu.async_copy(ocr_hbm.at[oidx_v.at[sl]],
                         rows_v.at[pl.ds(sbase + RPC, RPC), pl.ds(0, HID)], sem)

    def drain(sbase, sem):
        pltpu.make_async_copy(
            ans_hbm.at[aidx_v.at[pl.ds(0, RPC)]],
            rows_v.at[pl.ds(sbase, RPC), pl.ds(0, HID)], sem).wait()
        pltpu.make_async_copy(
            ans_hbm.at[aidx_v.at[pl.ds(0, RPC)]],
            rows_v.at[pl.ds(sbase + RPC, RPC), pl.ds(0, HID)], sem).wait()

    issue(NCH - 1, 0, sem0)
    issue(0, 32, sem1)

    def chunk_body(i, carry):
        ck = jnp.where(i == 0, NCH - 1, i - 1)
        slot = lax.rem(i, 2)
        sbase = slot * 32

        @pl.when(slot == 0)
        def _():
            drain(0, sem0)

        @pl.when(slot == 1)
        def _():
            drain(32, sem1)

        def quad_body(q, carry2):
            for h in range(4):
                r = q * 4 + h
                tok = r8 + ck * RPC + r
                idx_splat = plsc.load_gather(
                    idx_v, [jnp.broadcast_to(tok, (16,))])
                t_i32 = jnp.where(idx_splat >= ANS_NUM, 1, 0)
                xrow = jnp.broadcast_to(sbase + r, (16,)) + t_i32 * RPC

                zero = jnp.zeros((16,), jnp.float32)
                s = [zero] * 4
                s2 = [zero] * 4
                col = iota
                for j in range(NCHUNK):
                    x = plsc.load_gather(rows_v, [xrow, col])
                    col = col + 16
                    s[j % 4] = s[j % 4] + x
                    s2[j % 4] = s2[j % 4] + x * x
                mu = _lane_total((s[0] + s[1]) + (s[2] + s[3])) * (1.0 / HID)
                var = (_lane_total((s2[0] + s2[1]) + (s2[2] + s2[3]))
                       * (1.0 / HID) - mu * mu)
                rs = _rsqrt(var + 1e-12)
                murs = mu * rs

                col = iota
                pidx = t_i32 * HID + iota
                for j in range(NCHUNK):
                    x = plsc.load_gather(rows_v, [xrow, col])
                    wv = plsc.load_gather(pwt, [pidx])
                    bv = plsc.load_gather(pbt, [pidx])
                    stage[r, pl.ds(j * 16, 16)] = (x * rs - murs) * wv + bv
                    col = col + 16
                    pidx = pidx + 16
            return carry2

        lax.fori_loop(0, RPC // 4, quad_body, 0)
        pltpu.sync_copy(stage.at[pl.ds(0, RPC), pl.ds(0, HID)],
                        out_hbm.at[didx_v.at[ck]])

        cn = jnp.minimum(i + 1, NCH - 1)

        @pl.when(slot == 0)
        def _():
            issue(cn, 0, sem0)

        @pl.when(slot == 1)
        def _():
            issue(cn, 32, sem1)

        return carry

    lax.fori_loop(0, NCH, chunk_body, 0)
    drain(0, sem0)
    drain(32, sem1)


def kernel(ans_emb, ocr_emb, prev_inds, ans_w, ans_b, ocr_w, ocr_b, emb_w, emb_b, tt_table):
    batch = ocr_emb.shape[0]
    ocr_flat = jnp.transpose(ocr_emb, (1, 0, 2)).reshape(OCR_NUM * batch, HID)
    prev_flat = prev_inds.reshape(-1)
    tt_flat = tt_table.reshape(-1)
    mesh = plsc.VectorSubcoreMesh(core_axis_name="c", subcore_axis_name="s")
    run = functools.partial(
        pl.kernel,
        mesh=mesh,
        compiler_params=pltpu.CompilerParams(needs_layout_passes=False),
        out_type=jax.ShapeDtypeStruct((DEC_LEN * batch, HID), jnp.float32),
        scratch_types=[
            pltpu.VMEM((128,), jnp.int32),
            pltpu.VMEM((NCH * RPC,), jnp.int32),
            pltpu.VMEM((NCH * RPC,), jnp.int32),
            pltpu.VMEM((NCH, RPC), jnp.int32),
            pltpu.VMEM((64, 1024), jnp.float32),
            pltpu.VMEM((RPC, 1024), jnp.float32),
            pltpu.VMEM((2 * HID,), jnp.float32),
            pltpu.VMEM((2 * HID,), jnp.float32),
            pltpu.VMEM((2 * HID,), jnp.float32),
            pltpu.VMEM((HID,), jnp.float32),
            pltpu.VMEM((HID,), jnp.float32),
            pltpu.SemaphoreType.DMA,
            pltpu.SemaphoreType.DMA,
        ],
    )(_sc_body)
    out = run(ans_emb, ocr_flat, prev_flat, tt_flat,
              ans_w, ans_b, ocr_w, ocr_b, emb_w, emb_b)
    return jnp.transpose(out.reshape(DEC_LEN, batch, HID), (1, 0, 2))


# speedup vs baseline: 2.2789x; 1.4668x over previous
import functools

import jax
import jax.numpy as jnp
from jax import lax
from jax.experimental import pallas as pl
from jax.experimental.pallas import tpu as pltpu
from jax.experimental.pallas import tpu_sc as plsc

HID = 768
NCHUNK = HID // 16
ANS_NUM = 5000
OCR_NUM = 50
BATCH = 32
DEC_LEN = 100
RPC = 16
NCH = 7


def _rsqrt(x):
    xi = lax.bitcast_convert_type(x, jnp.int32)
    yi = jnp.int32(0x5F3759DF) - lax.shift_right_arithmetic(xi, 1)
    y = lax.bitcast_convert_type(yi, jnp.float32)
    for _ in range(3):
        y = y * (1.5 - 0.5 * x * y * y)
    return y


_GATHER_DNUMS = lax.GatherDimensionNumbers(
    offset_dims=(), collapsed_slice_dims=(0,), start_index_map=(0,))


def _permute(v, idx):
    return lax.gather(v, idx[:, None], _GATHER_DNUMS, slice_sizes=(1,),
                      mode=lax.GatherScatterMode.PROMISE_IN_BOUNDS)


def _lane_total(v):
    i = lax.iota(jnp.int32, 16)
    for st in (1, 2, 4, 8):
        v = v + _permute(v, i ^ st)
    return v


def _row_stats(read):
    zero = jnp.zeros((16,), jnp.float32)
    s = [zero] * 4
    s2 = [zero] * 4
    for j in range(NCHUNK):
        x = read(j)
        k = j % 4
        s[k] = s[k] + x
        s2[k] = s2[k] + x * x
    mu = _lane_total((s[0] + s[1]) + (s[2] + s[3])) * (1.0 / HID)
    var = _lane_total((s2[0] + s2[1]) + (s2[2] + s2[3])) * (1.0 / HID) - mu * mu
    return mu, _rsqrt(var + 1e-12)


def _sc_body(ans_hbm, ocr_hbm, prev_hbm, tt_hbm,
             ans_w_hbm, ans_b_hbm, ocr_w_hbm, ocr_b_hbm,
             emb_w_hbm, emb_b_hbm, out_hbm,
             idx_v, aidx_v, oidx_v, didx_v, rows_v, stage, tt_v, pwt, pbt,
             ew, eb, sem0, sem1):
    nc = 2
    wid = lax.axis_index("s") * nc + lax.axis_index("c")
    iota = lax.iota(jnp.int32, 16)
    zeros_i = jnp.zeros((16,), jnp.int32)

    tok0 = wid * DEC_LEN
    r8 = lax.bitwise_and(tok0, 7)
    idx_v[pl.ds(96, 16)] = zeros_i
    idx_v[pl.ds(112, 16)] = zeros_i
    abase = pl.multiple_of(tok0 - r8, 8)
    pltpu.sync_copy(prev_hbm.at[pl.ds(abase, 104)], idx_v.at[pl.ds(0, 104)])

    for k in range(NCH):
        v = plsc.load_gather(idx_v, [jnp.broadcast_to(r8 + k * 16, (16,)) + iota])
        t = v >= ANS_NUM
        aidx_v[pl.ds(k * 16, 16)] = jnp.where(t, 0, v)
        oidx_v[pl.ds(k * 16, 16)] = jnp.where(t, (v - ANS_NUM) * BATCH + wid, wid)
        tokv = iota + (k * 16)
        if k == NCH - 1:
            tokv = jnp.where(iota < 4, tokv, iota - 4)
        didx_v[k, :] = tokv * BATCH + wid

    pltpu.sync_copy(ans_w_hbm, pwt.at[0])
    pltpu.sync_copy(ocr_w_hbm, pwt.at[1])
    pltpu.sync_copy(ans_b_hbm, pbt.at[0])
    pltpu.sync_copy(ocr_b_hbm, pbt.at[1])
    pltpu.sync_copy(emb_w_hbm, ew)
    pltpu.sync_copy(emb_b_hbm, eb)
    pltpu.sync_copy(tt_hbm.at[pl.ds(0, 2 * HID)], tt_v)

    mu0, rs0 = _row_stats(lambda j: tt_v[pl.ds(j * 16, 16)])
    mu1, rs1 = _row_stats(lambda j: tt_v[pl.ds(HID + j * 16, 16)])
    for j in range(NCHUNK):
        sl = pl.ds(j * 16, 16)
        tte0 = (tt_v[sl] - mu0) * rs0 * ew[sl] + eb[sl]
        pbt[0, sl] = pbt[0, sl] + tte0
        tte1 = (tt_v[pl.ds(HID + j * 16, 16)] - mu1) * rs1 * ew[sl] + eb[sl]
        pbt[1, sl] = pbt[1, sl] + tte1

    def issue(ck, slot):
        sem = sem0 if slot == 0 else sem1
        sl = pl.ds(ck * RPC, RPC)
        ca = pltpu.async_copy(ans_hbm.at[aidx_v.at[sl]],
                              rows_v.at[pl.ds(slot * 32, RPC)], sem)
        co = pltpu.async_copy(ocr_hbm.at[oidx_v.at[sl]],
                              rows_v.at[pl.ds(slot * 32 + RPC, RPC)], sem)
        return ca, co

    def compute_chunk(ck, slot):
        def row_body(r, carry):
            tok = r8 + ck * RPC + r
            idx_splat = plsc.load_gather(idx_v, [jnp.broadcast_to(tok, (16,))])
            t_i32 = jnp.where(idx_splat >= ANS_NUM, 1, 0)
            xrow = jnp.broadcast_to(slot * 32 + r, (16,)) + t_i32 * RPC

            cols = [iota + (j * 16) for j in range(NCHUNK)]
            mu, rs = _row_stats(
                lambda j: plsc.load_gather(rows_v, [xrow, cols[j]]))

            for j in range(NCHUNK):
                x = plsc.load_gather(rows_v, [xrow, cols[j]])
                wv = plsc.load_gather(pwt, [t_i32, cols[j]])
                bv = plsc.load_gather(pbt, [t_i32, cols[j]])
                a = wv * rs
                cc = bv - mu * a
                stage[r, pl.ds(j * 16, 16)] = x * a + cc
            return carry

        lax.fori_loop(0, RPC, row_body, 0)
        pltpu.sync_copy(stage, out_hbm.at[didx_v.at[ck]])

    order = [NCH - 1] + list(range(NCH - 1))
    pending = {0: issue(order[0], 0), 1: issue(order[1], 1)}
    for i, ck in enumerate(order):
        slot = i % 2
        ca, co = pending.pop(i)
        ca.wait()
        co.wait()
        compute_chunk(ck, slot)
        if i + 2 < NCH:
            pending[i + 2] = issue(order[i + 2], slot)


def kernel(ans_emb, ocr_emb, prev_inds, ans_w, ans_b, ocr_w, ocr_b, emb_w, emb_b, tt_table):
    batch = ocr_emb.shape[0]
    ocr_flat = jnp.transpose(ocr_emb, (1, 0, 2)).reshape(OCR_NUM * batch, HID)
    prev_flat = prev_inds.reshape(-1)
    tt_flat = tt_table.reshape(-1)
    mesh = plsc.VectorSubcoreMesh(core_axis_name="c", subcore_axis_name="s")
    run = functools.partial(
        pl.kernel,
        mesh=mesh,
        compiler_params=pltpu.CompilerParams(needs_layout_passes=False),
        out_type=jax.ShapeDtypeStruct((DEC_LEN * batch, HID), jnp.float32),
        scratch_types=[
            pltpu.VMEM((128,), jnp.int32),
            pltpu.VMEM((NCH * RPC,), jnp.int32),
            pltpu.VMEM((NCH * RPC,), jnp.int32),
            pltpu.VMEM((NCH, RPC), jnp.int32),
            pltpu.VMEM((64, HID), jnp.float32),
            pltpu.VMEM((RPC, HID), jnp.float32),
            pltpu.VMEM((2 * HID,), jnp.float32),
            pltpu.VMEM((2, HID), jnp.float32),
            pltpu.VMEM((2, HID), jnp.float32),
            pltpu.VMEM((HID,), jnp.float32),
            pltpu.VMEM((HID,), jnp.float32),
            pltpu.SemaphoreType.DMA,
            pltpu.SemaphoreType.DMA,
        ],
    )(_sc_body)
    out = run(ans_emb, ocr_flat, prev_flat, tt_flat,
              ans_w, ans_b, ocr_w, ocr_b, emb_w, emb_b)
    return jnp.transpose(out.reshape(DEC_LEN, batch, HID), (1, 0, 2))


# speedup vs baseline: 2.3666x; 1.0385x over previous
import functools

import jax
import jax.numpy as jnp
from jax import lax
from jax.experimental import pallas as pl
from jax.experimental.pallas import tpu as pltpu
from jax.experimental.pallas import tpu_sc as plsc

HID = 768
NCHUNK = HID // 16
ANS_NUM = 5000
OCR_NUM = 50
BATCH = 32
DEC_LEN = 100
RPC = 16
NCH = 7


def _rsqrt(x):
    xi = lax.bitcast_convert_type(x, jnp.int32)
    yi = jnp.int32(0x5F3759DF) - lax.shift_right_arithmetic(xi, 1)
    y = lax.bitcast_convert_type(yi, jnp.float32)
    for _ in range(3):
        y = y * (1.5 - 0.5 * x * y * y)
    return y


_GATHER_DNUMS = lax.GatherDimensionNumbers(
    offset_dims=(), collapsed_slice_dims=(0,), start_index_map=(0,))


def _permute(v, idx):
    return lax.gather(v, idx[:, None], _GATHER_DNUMS, slice_sizes=(1,),
                      mode=lax.GatherScatterMode.PROMISE_IN_BOUNDS)


def _lane_total(v):
    i = lax.iota(jnp.int32, 16)
    for st in (1, 2, 4, 8):
        v = v + _permute(v, i ^ st)
    return v


def _row_stats(read):
    zero = jnp.zeros((16,), jnp.float32)
    s = [zero] * 4
    s2 = [zero] * 4
    for j in range(NCHUNK):
        x = read(j)
        k = j % 4
        s[k] = s[k] + x
        s2[k] = s2[k] + x * x
    mu = _lane_total((s[0] + s[1]) + (s[2] + s[3])) * (1.0 / HID)
    var = _lane_total((s2[0] + s2[1]) + (s2[2] + s2[3])) * (1.0 / HID) - mu * mu
    return mu, _rsqrt(var + 1e-12)


def _sc_body(ans_hbm, ocr_hbm, prev_hbm, tt_hbm,
             ans_w_hbm, ans_b_hbm, ocr_w_hbm, ocr_b_hbm,
             emb_w_hbm, emb_b_hbm, out_hbm,
             idx_v, aidx_v, oidx_v, didx_v, rows_v, stage0, stage1, tt_v,
             pwt, pbt, ew, eb, sem0, sem1, semo0, semo1):
    nc = 2
    wid = lax.axis_index("s") * nc + lax.axis_index("c")
    iota = lax.iota(jnp.int32, 16)
    zeros_i = jnp.zeros((16,), jnp.int32)

    tok0 = wid * DEC_LEN
    r8 = lax.bitwise_and(tok0, 7)
    idx_v[pl.ds(96, 16)] = zeros_i
    idx_v[pl.ds(112, 16)] = zeros_i
    abase = pl.multiple_of(tok0 - r8, 8)
    pltpu.sync_copy(prev_hbm.at[pl.ds(abase, 104)], idx_v.at[pl.ds(0, 104)])

    for k in range(NCH):
        v = plsc.load_gather(idx_v, [jnp.broadcast_to(r8 + k * 16, (16,)) + iota])
        t = v >= ANS_NUM
        aidx_v[pl.ds(k * 16, 16)] = jnp.where(t, 0, v)
        oidx_v[pl.ds(k * 16, 16)] = jnp.where(t, (v - ANS_NUM) * BATCH + wid, wid)
        tokv = iota + (k * 16)
        if k == NCH - 1:
            tokv = jnp.where(iota < 4, tokv, iota - 4)
        didx_v[k, :] = tokv * BATCH + wid

    pltpu.sync_copy(ans_w_hbm, pwt.at[0])
    pltpu.sync_copy(ocr_w_hbm, pwt.at[1])
    pltpu.sync_copy(ans_b_hbm, pbt.at[0])
    pltpu.sync_copy(ocr_b_hbm, pbt.at[1])
    pltpu.sync_copy(emb_w_hbm, ew)
    pltpu.sync_copy(emb_b_hbm, eb)
    pltpu.sync_copy(tt_hbm.at[pl.ds(0, 2 * HID)], tt_v)

    mu0, rs0 = _row_stats(lambda j: tt_v[pl.ds(j * 16, 16)])
    mu1, rs1 = _row_stats(lambda j: tt_v[pl.ds(HID + j * 16, 16)])
    for j in range(NCHUNK):
        sl = pl.ds(j * 16, 16)
        tte0 = (tt_v[sl] - mu0) * rs0 * ew[sl] + eb[sl]
        pbt[0, sl] = pbt[0, sl] + tte0
        tte1 = (tt_v[pl.ds(HID + j * 16, 16)] - mu1) * rs1 * ew[sl] + eb[sl]
        pbt[1, sl] = pbt[1, sl] + tte1

    def issue(ck, slot):
        sem = sem0 if slot == 0 else sem1
        sl = pl.ds(ck * RPC, RPC)
        ca = pltpu.async_copy(ans_hbm.at[aidx_v.at[sl]],
                              rows_v.at[pl.ds(slot * 32, RPC)], sem)
        co = pltpu.async_copy(ocr_hbm.at[oidx_v.at[sl]],
                              rows_v.at[pl.ds(slot * 32 + RPC, RPC)], sem)
        return ca, co

    def compute_chunk(ck, slot, sync_out):
        stage = stage0 if slot == 0 else stage1

        def row_body(r, carry):
            tok = r8 + ck * RPC + r
            idx_splat = plsc.load_gather(idx_v, [jnp.broadcast_to(tok, (16,))])
            t_i32 = jnp.where(idx_splat >= ANS_NUM, 1, 0)
            xrow = jnp.broadcast_to(slot * 32 + r, (16,)) + t_i32 * RPC

            cols = [iota + (j * 16) for j in range(NCHUNK)]
            mu, rs = _row_stats(
                lambda j: plsc.load_gather(rows_v, [xrow, cols[j]]))

            for j in range(NCHUNK):
                x = plsc.load_gather(rows_v, [xrow, cols[j]])
                wv = plsc.load_gather(pwt, [t_i32, cols[j]])
                bv = plsc.load_gather(pbt, [t_i32, cols[j]])
                a = wv * rs
                cc = bv - mu * a
                stage[r, pl.ds(j * 16, 16)] = x * a + cc
            return carry

        lax.fori_loop(0, RPC, row_body, 0)
        if sync_out:
            pltpu.sync_copy(stage, out_hbm.at[didx_v.at[ck]])
            return None
        semo = semo0 if slot == 0 else semo1
        return pltpu.async_copy(stage, out_hbm.at[didx_v.at[ck]], semo)

    order = [NCH - 1] + list(range(NCH - 1))
    pending = {0: issue(order[0], 0), 1: issue(order[1], 1)}
    out_pending = {0: None, 1: None}
    for i, ck in enumerate(order):
        slot = i % 2
        ca, co = pending.pop(i)
        ca.wait()
        co.wait()
        if out_pending[slot] is not None:
            out_pending[slot].wait()
        out_pending[slot] = compute_chunk(ck, slot, sync_out=(i == 0))
        if i + 2 < NCH:
            pending[i + 2] = issue(order[i + 2], slot)
    for slot in (0, 1):
        if out_pending[slot] is not None:
            out_pending[slot].wait()


def kernel(ans_emb, ocr_emb, prev_inds, ans_w, ans_b, ocr_w, ocr_b, emb_w, emb_b, tt_table):
    batch = ocr_emb.shape[0]
    ocr_flat = jnp.transpose(ocr_emb, (1, 0, 2)).reshape(OCR_NUM * batch, HID)
    prev_flat = prev_inds.reshape(-1)
    tt_flat = tt_table.reshape(-1)
    mesh = plsc.VectorSubcoreMesh(core_axis_name="c", subcore_axis_name="s")
    run = functools.partial(
        pl.kernel,
        mesh=mesh,
        compiler_params=pltpu.CompilerParams(needs_layout_passes=False),
        out_type=jax.ShapeDtypeStruct((DEC_LEN * batch, HID), jnp.float32),
        scratch_types=[
            pltpu.VMEM((128,), jnp.int32),
            pltpu.VMEM((NCH * RPC,), jnp.int32),
            pltpu.VMEM((NCH * RPC,), jnp.int32),
            pltpu.VMEM((NCH, RPC), jnp.int32),
            pltpu.VMEM((64, HID), jnp.float32),
            pltpu.VMEM((RPC, HID), jnp.float32),
            pltpu.VMEM((RPC, HID), jnp.float32),
            pltpu.VMEM((2 * HID,), jnp.float32),
            pltpu.VMEM((2, HID), jnp.float32),
            pltpu.VMEM((2, HID), jnp.float32),
            pltpu.VMEM((HID,), jnp.float32),
            pltpu.VMEM((HID,), jnp.float32),
            pltpu.SemaphoreType.DMA,
            pltpu.SemaphoreType.DMA,
            pltpu.SemaphoreType.DMA,
            pltpu.SemaphoreType.DMA,
        ],
    )(_sc_body)
    out = run(ans_emb, ocr_flat, prev_flat, tt_flat,
              ans_w, ans_b, ocr_w, ocr_b, emb_w, emb_b)
    return jnp.transpose(out.reshape(DEC_LEN, batch, HID), (1, 0, 2))
